# baseline (speedup 1.0000x reference)
import functools

import jax
import jax.numpy as jnp
from jax import lax
from jax.experimental import pallas as pl
from jax.experimental.pallas import tpu as pltpu
from jax.experimental.pallas import tpu_sc as plsc

LVL = 4
BATCH = 2
LB = LVL * BATCH
C = 96
CPAD = 128
H = 224
W = 224
HW = H * W
NQ = 8192
OUTC = LVL * C

NPAIR = 2
NSPLIT = LVL // NPAIR
PB = NPAIR * BATCH

NC = 2
NS = 16
LANES = 16
NW = NC * NS
QPW = NQ // NW
CHUNK = 32
NCHUNK = QPW // CHUNK
IDXC = 4 * CHUNK
NBLK = QPW // LANES

TBLK = 25088
NTBLK = HW // TBLK

_SPLAT_DNUMS = jax.lax.GatherDimensionNumbers(
    offset_dims=(), collapsed_slice_dims=(0,), start_index_map=(0,))


HB = 56
NHB = H // HB


def _tr_body(x_ref, o_ref):
    eye = (lax.broadcasted_iota(jnp.int32, (C, CPAD), 0)
           == lax.broadcasted_iota(jnp.int32, (C, CPAD), 1)
           ).astype(jnp.float32)
    x = x_ref[0, 0]
    for h in range(HB):
        o_ref[pl.ds(h * W, W), :] = lax.dot_general(
            x[:, h, :], eye, (((0,), (0,)), ((), ())),
            preferred_element_type=jnp.float32)


def _build_tables(feats, p):
    return pl.pallas_call(
        _tr_body,
        out_shape=jax.ShapeDtypeStruct((PB * HW, CPAD), jnp.float32),
        grid=(PB, NHB),
        in_specs=[pl.BlockSpec(
            (1, 1, C, HB, W),
            lambda i, j: ((p * PB + i) // BATCH, (p * PB + i) % BATCH,
                          0, j, 0))],
        out_specs=pl.BlockSpec((HB * W, CPAD),
                               lambda i, j: (i * NHB + j, 0)),
    )(feats)


def _sc_body(tables, xs, ys, out, x_v, y_v, w_v, base_v, idx_v, rows_v,
             out_v, *sems):
    wid = lax.axis_index("s") * NC + lax.axis_index("c")
    qbase = wid * QPW
    iota = lax.iota(jnp.int32, LANES)

    for b in range(BATCH):
        cx = pltpu.async_copy(xs.at[pl.ds(b * NQ + qbase, QPW)], x_v,
                              sems[4])
        cy = pltpu.async_copy(ys.at[pl.ds(b * NQ + qbase, QPW)], y_v,
                              sems[5])
        cx.wait()
        cy.wait()

        def blk(i, _):
            q0 = i * LANES
            xv = x_v[pl.ds(q0, LANES)]
            yv = y_v[pl.ds(q0, LANES)]
            xi = jnp.clip(xv.astype(jnp.int32), 0, W - 2)
            yi = jnp.clip(yv.astype(jnp.int32), 0, H - 2)
            fx = xv - xi.astype(jnp.float32)
            fy = yv - yi.astype(jnp.float32)
            gx = 1.0 - fx
            gy = 1.0 - fy
            w_v[pl.ds(0 * QPW + q0, LANES)] = gy * gx
            w_v[pl.ds(1 * QPW + q0, LANES)] = gy * fx
            w_v[pl.ds(2 * QPW + q0, LANES)] = fy * gx
            w_v[pl.ds(3 * QPW + q0, LANES)] = fy * fx
            base = yi * W + xi + (b * HW)
            ch = i // 2
            h = i % 2
            d0 = ch * IDXC + h * LANES
            for k, delta in enumerate((0, 1, W, W + 1)):
                base_v[pl.ds(d0 + k * CHUNK, LANES)] = base + delta
            return 0

        lax.fori_loop(0, NBLK, blk, 0)

        def lvl(j, _):
            v = base_v[pl.ds(j * LANES, LANES)]
            for l in range(NPAIR):
                idx_v[pl.ds(l * (NCHUNK * IDXC) + j * LANES, LANES)] = (
                    v + l * (BATCH * HW))
            return 0

        lax.fori_loop(0, NCHUNK * IDXC // LANES, lvl, 0)

        def issue(ch, l):
            idx_ref = idx_v.at[pl.ds(l * (NCHUNK * IDXC) + ch * IDXC, IDXC)]
            return pltpu.async_copy(
                tables.at[idx_ref],
                rows_v.at[pl.ds(l * IDXC, IDXC)],
                sems[l])

        for l in range(NPAIR):
            issue(0, l)

        def out_desc(ch):
            return pltpu.make_async_copy(
                out_v.at[0], out.at[b, pl.ds(qbase + ch * CHUNK, CHUNK)],
                sems[6])

        def round_(ch, _):
            @pl.when(ch >= 1)
            def _():
                out_desc(0).wait()

            for l in range(NPAIR):
                pltpu.make_async_copy(
                    tables.at[idx_v.at[pl.ds(0, IDXC)]],
                    rows_v.at[pl.ds(l * IDXC, IDXC)],
                    sems[l]).wait()

            wbase = ch * CHUNK

            def qloop(i, _):
                for u in range(2):
                    q = i * 2 + u
                    qb = q // LANES
                    qm = lax.broadcast(q % LANES, (LANES,))
                    ws = []
                    for k in range(4):
                        wv = w_v[pl.ds(wbase + k * QPW + qb * LANES, LANES)]
                        ws.append(lax.gather(
                            wv, qm[:, None], _SPLAT_DNUMS, slice_sizes=(1,),
                            mode=lax.GatherScatterMode.PROMISE_IN_BOUNDS))
                    for l in range(NPAIR):
                        for c6 in range(C // LANES):
                            acc = None
                            for k in range(4):
                                g = rows_v[l * IDXC + k * CHUNK + q,
                                           pl.ds(c6 * LANES, LANES)]
                                t = g * ws[k]
                                acc = t if acc is None else acc + t
                            out_v[0, q,
                                  pl.ds(l * C + c6 * LANES, LANES)] = acc
                return 0

            lax.fori_loop(0, CHUNK // 2, qloop, 0)

            for l in range(NPAIR):
                @pl.when(ch + 1 < NCHUNK)
                def _():
                    issue(ch + 1, l)

            out_desc(ch).start()
            return 0

        lax.fori_loop(0, NCHUNK, round_, 0)
        out_desc(NCHUNK - 1).wait()


@jax.jit
def _sc_call(tables, xs, ys):
    mesh = plsc.VectorSubcoreMesh(core_axis_name="c", subcore_axis_name="s")
    return pl.kernel(
        _sc_body,
        out_type=jax.ShapeDtypeStruct((BATCH, NQ, NPAIR * C), jnp.float32),
        mesh=mesh,
        scratch_types=[
            pltpu.VMEM((QPW,), jnp.float32),
            pltpu.VMEM((QPW,), jnp.float32),
            pltpu.VMEM((4 * QPW,), jnp.float32),
            pltpu.VMEM((NCHUNK * IDXC,), jnp.int32),
            pltpu.VMEM((NPAIR * NCHUNK * IDXC,), jnp.int32),
            pltpu.VMEM((NPAIR * IDXC, CPAD), jnp.float32),
            pltpu.VMEM((1, CHUNK, NPAIR * C), jnp.float32),
        ] + [pltpu.SemaphoreType.DMA] * 8,
    )(tables, xs, ys)


def kernel(input_feats, input_coords, input_size):
    xs = (input_coords[:, :, 0] * ((W - 1.0) / input_size)).reshape(-1)
    ys = (input_coords[:, :, 1] * ((H - 1.0) / input_size)).reshape(-1)
    outs = []
    for p in range(NSPLIT):
        tables = _build_tables(input_feats, p)
        outs.append(_sc_call(tables, xs, ys))
    out = jnp.concatenate(outs, axis=-1)
    return (out[0], out[1])


# speedup vs baseline: 1.1626x; 1.1626x over previous
import functools

import jax
import jax.numpy as jnp
from jax import lax
from jax.experimental import pallas as pl
from jax.experimental.pallas import tpu as pltpu
from jax.experimental.pallas import tpu_sc as plsc

LVL = 4
BATCH = 2
LB = LVL * BATCH
C = 96
CPAD = 128
H = 224
W = 224
HW = H * W
NQ = 8192
OUTC = LVL * C

NPAIR = 2
NSPLIT = LVL // NPAIR
PB = NPAIR * BATCH

NC = 2
NS = 16
LANES = 16
NW = NC * NS
QPW = NQ // NW
CHUNK = 32
NCHUNK = QPW // CHUNK
IDXC = 4 * CHUNK
NBLK = QPW // LANES

TBLK = 25088
NTBLK = HW // TBLK

_SPLAT_DNUMS = jax.lax.GatherDimensionNumbers(
    offset_dims=(), collapsed_slice_dims=(0,), start_index_map=(0,))


HB = 56
NHB = H // HB


def _tr_body(x_ref, o_ref):
    eye = (lax.broadcasted_iota(jnp.int32, (C, CPAD), 0)
           == lax.broadcasted_iota(jnp.int32, (C, CPAD), 1)
           ).astype(jnp.float32)
    x = x_ref[0, 0]
    for h in range(HB):
        o_ref[pl.ds(h * W, W), :] = lax.dot_general(
            x[:, h, :], eye, (((0,), (0,)), ((), ())),
            preferred_element_type=jnp.float32)


def _build_tables(feats, p):
    return pl.pallas_call(
        _tr_body,
        out_shape=jax.ShapeDtypeStruct((PB * HW, CPAD), jnp.float32),
        grid=(PB, NHB),
        in_specs=[pl.BlockSpec(
            (1, 1, C, HB, W),
            lambda i, j: ((p * PB + i) // BATCH, (p * PB + i) % BATCH,
                          0, j, 0))],
        out_specs=pl.BlockSpec((HB * W, CPAD),
                               lambda i, j: (i * NHB + j, 0)),
    )(feats)


def _sc_body(tables, xs, ys, out, x_v, y_v, w_v, base_v, idx_v, rows_v,
             out_v, *sems):
    wid = lax.axis_index("s") * NC + lax.axis_index("c")
    qbase = wid * QPW
    iota = lax.iota(jnp.int32, LANES)

    for b in range(BATCH):
        cx = pltpu.async_copy(xs.at[pl.ds(b * NQ + qbase, QPW)], x_v,
                              sems[4])
        cy = pltpu.async_copy(ys.at[pl.ds(b * NQ + qbase, QPW)], y_v,
                              sems[5])
        cx.wait()
        cy.wait()

        def blk(i, _):
            q0 = i * LANES
            xv = x_v[pl.ds(q0, LANES)]
            yv = y_v[pl.ds(q0, LANES)]
            xi = jnp.clip(xv.astype(jnp.int32), 0, W - 2)
            yi = jnp.clip(yv.astype(jnp.int32), 0, H - 2)
            fx = xv - xi.astype(jnp.float32)
            fy = yv - yi.astype(jnp.float32)
            gx = 1.0 - fx
            gy = 1.0 - fy
            w_v[pl.ds(0 * QPW + q0, LANES)] = gy * gx
            w_v[pl.ds(1 * QPW + q0, LANES)] = gy * fx
            w_v[pl.ds(2 * QPW + q0, LANES)] = fy * gx
            w_v[pl.ds(3 * QPW + q0, LANES)] = fy * fx
            base = yi * W + xi + (b * HW)
            ch = i // 2
            h = i % 2
            d0 = ch * IDXC + h * LANES
            for k, delta in enumerate((0, 1, W, W + 1)):
                base_v[pl.ds(d0 + k * CHUNK, LANES)] = base + delta
            return 0

        lax.fori_loop(0, NBLK, blk, 0)

        def lvl(j, _):
            v = base_v[pl.ds(j * LANES, LANES)]
            for l in range(NPAIR):
                idx_v[pl.ds(l * (NCHUNK * IDXC) + j * LANES, LANES)] = (
                    v + l * (BATCH * HW))
            return 0

        lax.fori_loop(0, NCHUNK * IDXC // LANES, lvl, 0)

        def issue(ch, l, s):
            idx_ref = idx_v.at[pl.ds(l * (NCHUNK * IDXC) + ch * IDXC, IDXC)]
            return pltpu.async_copy(
                tables.at[idx_ref],
                rows_v.at[pl.ds((l * 2 + s) * IDXC, IDXC)],
                sems[l * 2 + s])

        for l in range(NPAIR):
            for s in range(2):
                issue(s, l, s)

        def combine(ch, s):
            def qloop(i, _):
                for u in range(2):
                    q = i * 2 + u
                    qb = q // LANES
                    qm = lax.broadcast(q % LANES, (LANES,))
                    ws = []
                    for k in range(4):
                        wv = w_v[pl.ds(k * QPW + ch * CHUNK + qb * LANES,
                                       LANES)]
                        ws.append(lax.gather(
                            wv, qm[:, None], _SPLAT_DNUMS, slice_sizes=(1,),
                            mode=lax.GatherScatterMode.PROMISE_IN_BOUNDS))
                    for l in range(NPAIR):
                        r0 = (l * 2 + s) * IDXC
                        for c6 in range(C // LANES):
                            acc = None
                            for k in range(4):
                                g = rows_v[r0 + k * CHUNK + q,
                                           pl.ds(c6 * LANES, LANES)]
                                t = g * ws[k]
                                acc = t if acc is None else acc + t
                            out_v[s, q, pl.ds(l * C + c6 * LANES, LANES)] = (
                                acc)
                return 0

            lax.fori_loop(0, CHUNK // 2, qloop, 0)

        def out_desc(b_, ch, s):
            return pltpu.make_async_copy(
                out_v.at[s], out.at[b_, pl.ds(qbase + ch * CHUNK, CHUNK)],
                sems[6 + s])

        def round2_(ch2, _):
            for s in range(2):
                ch = ch2 * 2 + s

                @pl.when(ch >= 2)
                def _():
                    out_desc(b, ch - 2, s).wait()

                for l in range(NPAIR):
                    pltpu.make_async_copy(
                        tables.at[idx_v.at[pl.ds(0, IDXC)]],
                        rows_v.at[pl.ds((l * 2 + s) * IDXC, IDXC)],
                        sems[l * 2 + s]).wait()
                combine(ch, s)
                for l in range(NPAIR):
                    @pl.when(ch + 2 < NCHUNK)
                    def _():
                        issue(ch + 2, l, s)

                out_desc(b, ch, s).start()
            return 0

        lax.fori_loop(0, NCHUNK // 2, round2_, 0)
        for s in range(2):
            out_desc(b, NCHUNK - 2 + s, s).wait()


@jax.jit
def _sc_call(tables, xs, ys):
    mesh = plsc.VectorSubcoreMesh(core_axis_name="c", subcore_axis_name="s")
    return pl.kernel(
        _sc_body,
        out_type=jax.ShapeDtypeStruct((BATCH, NQ, NPAIR * C), jnp.float32),
        mesh=mesh,
        scratch_types=[
            pltpu.VMEM((QPW,), jnp.float32),
            pltpu.VMEM((QPW,), jnp.float32),
            pltpu.VMEM((4 * QPW,), jnp.float32),
            pltpu.VMEM((NCHUNK * IDXC,), jnp.int32),
            pltpu.VMEM((NPAIR * NCHUNK * IDXC,), jnp.int32),
            pltpu.VMEM((NPAIR * 2 * IDXC, CPAD), jnp.float32),
            pltpu.VMEM((2, CHUNK, NPAIR * C), jnp.float32),
        ] + [pltpu.SemaphoreType.DMA] * 8,
    )(tables, xs, ys)


def kernel(input_feats, input_coords, input_size):
    xs = (input_coords[:, :, 0] * ((W - 1.0) / input_size)).reshape(-1)
    ys = (input_coords[:, :, 1] * ((H - 1.0) / input_size)).reshape(-1)
    outs = []
    for p in range(NSPLIT):
        tables = _build_tables(input_feats, p)
        outs.append(_sc_call(tables, xs, ys))
    out = jnp.concatenate(outs, axis=-1)
    return (out[0], out[1])
